# baseline (speedup 1.0000x reference)
import functools

import jax
import jax.numpy as jnp
from jax import lax
from jax.experimental import pallas as pl
from jax.experimental.pallas import tpu as pltpu
from jax.experimental.pallas import tpu_sc as plsc

N = 10000
E = 320000
D = 128
NC = 2
NS = 16
NW = NC * NS
CHUNK = 128
ROWS = E // CHUNK
PAIRS = ROWS // 2
PPW = PAIRS // NW
ZROWS = 10112
ZSTRIPE = ZROWS // NS
DEGW = 128
RBLK = 10000
GRID = N // RBLK

_sc_mesh = plsc.VectorSubcoreMesh(
    core_axis_name="c", subcore_axis_name="s", num_cores=NC, num_subcores=NS
)


def _init_stripe(zer_hbm, zbuf, acc, s):
    pltpu.sync_copy(zer_hbm, zbuf)
    for i in range(ZSTRIPE // CHUNK):
        pltpu.sync_copy(zbuf, acc.at[pl.ds(s * ZSTRIPE + i * CHUNK, CHUNK)])
    rem = ZSTRIPE % CHUNK
    if rem:
        pltpu.sync_copy(
            zbuf.at[pl.ds(0, rem)],
            acc.at[pl.ds(s * ZSTRIPE + ZSTRIPE - rem, rem)],
        )


@functools.partial(
    pl.kernel,
    out_type=jax.ShapeDtypeStruct((NC, ZROWS, DEGW), jnp.float32),
    mesh=_sc_mesh,
    scratch_types=[
        pltpu.VMEM((2, CHUNK), jnp.int32),
        pltpu.VMEM((2, CHUNK), jnp.int32),
        pltpu.VMEM((CHUNK, DEGW), jnp.float32),
        pltpu.VMEM((CHUNK, DEGW), jnp.float32),
        pltpu.VMEM_SHARED((ZROWS, DEGW), jnp.float32),
        pltpu.SemaphoreType.DMA,
    ],
)
def _deg_kernel(idx_hbm, ones_hbm, zer_hbm, out_hbm,
                q0, q1, onesv, zbuf, acc, sem_i):
    c = lax.axis_index("c")
    s = lax.axis_index("s")
    wid = c * NS + s
    _init_stripe(zer_hbm, zbuf, acc, s)
    pltpu.sync_copy(ones_hbm, onesv)
    plsc.subcore_barrier()

    p0 = wid * PPW

    def load(p, q):
        pltpu.async_copy(idx_hbm.at[1, pl.ds(p * 2 * CHUNK, CHUNK)],
                         q.at[0], sem_i)
        pltpu.async_copy(idx_hbm.at[1, pl.ds(p * 2 * CHUNK + CHUNK, CHUNK)],
                         q.at[1], sem_i)

    def drain(p, q):
        pltpu.make_async_copy(idx_hbm.at[1, pl.ds(p * 2 * CHUNK, CHUNK)],
                              q.at[0], sem_i).wait()
        pltpu.make_async_copy(idx_hbm.at[1, pl.ds(p * 2 * CHUNK, CHUNK)],
                              q.at[1], sem_i).wait()

    def scat(q):
        pltpu.sync_copy(onesv, acc.at[q.at[0]], add=True)
        pltpu.sync_copy(onesv, acc.at[q.at[1]], add=True)

    load(p0, q0)
    drain(p0, q0)

    def dbody(k, carry):
        j0 = p0 + 2 * k
        load(j0 + 1, q1)
        scat(q0)
        drain(j0 + 1, q1)
        load(j0 + 2, q0)
        scat(q1)
        drain(j0 + 2, q0)
        return carry

    lax.fori_loop(0, (PPW - 1) // 2, dbody, 0)
    scat(q0)

    @pl.when(wid < 2)
    def _():
        load(NW * PPW + wid, q1)
        drain(NW * PPW + wid, q1)
        scat(q1)

    plsc.subcore_barrier()
    pltpu.sync_copy(
        acc.at[pl.ds(s * ZSTRIPE, ZSTRIPE)],
        out_hbm.at[c, pl.ds(s * ZSTRIPE, ZSTRIPE)],
    )


@functools.partial(
    pl.kernel,
    out_type=jax.ShapeDtypeStruct((NC, ZROWS, D), jnp.float32),
    mesh=_sc_mesh,
    scratch_types=[
        pltpu.VMEM((2, 2, CHUNK), jnp.int32),
        pltpu.VMEM((2, 2, CHUNK), jnp.int32),
        pltpu.VMEM((CHUNK, D), jnp.float32),
        pltpu.VMEM((CHUNK, D), jnp.float32),
        pltpu.VMEM_SHARED((ZROWS, D), jnp.float32),
        pltpu.SemaphoreType.DMA,
        pltpu.SemaphoreType.DMA,
        pltpu.SemaphoreType.DMA,
        pltpu.SemaphoreType.DMA,
    ],
)
def _edge_kernel(y_hbm, idx_hbm, zer_hbm, out_hbm,
                 q0, q1, rows_a, rows_b, acc, sem_a, sem_b, sem_i1, sem_i0):
    c = lax.axis_index("c")
    s = lax.axis_index("s")
    wid = c * NS + s
    _init_stripe(zer_hbm, rows_a, acc, s)
    plsc.subcore_barrier()

    p0 = wid * PPW

    def halfstep(qc, qn, pn, sem_in):
        pltpu.async_copy(y_hbm.at[qc.at[0, 1]], rows_b, sem_b)
        pltpu.async_copy(idx_hbm.at[0, pl.ds(pn * 2 * CHUNK, CHUNK)],
                         qn.at[0, 0], sem_in)
        pltpu.async_copy(idx_hbm.at[0, pl.ds(pn * 2 * CHUNK + CHUNK, CHUNK)],
                         qn.at[0, 1], sem_in)
        pltpu.async_copy(idx_hbm.at[1, pl.ds(pn * 2 * CHUNK, CHUNK)],
                         qn.at[1, 0], sem_in)
        pltpu.async_copy(idx_hbm.at[1, pl.ds(pn * 2 * CHUNK + CHUNK, CHUNK)],
                         qn.at[1, 1], sem_in)
        pltpu.make_async_copy(y_hbm.at[qc.at[0, 0]], rows_a, sem_a).wait()
        pltpu.sync_copy(rows_a, acc.at[qc.at[1, 0]], add=True)
        for _k in range(4):
            pltpu.make_async_copy(idx_hbm.at[0, pl.ds(pn * 2 * CHUNK, CHUNK)],
                                  qn.at[0, 0], sem_in).wait()
        pltpu.async_copy(y_hbm.at[qn.at[0, 0]], rows_a, sem_a)
        pltpu.make_async_copy(y_hbm.at[qc.at[0, 1]], rows_b, sem_b).wait()
        pltpu.sync_copy(rows_b, acc.at[qc.at[1, 1]], add=True)

    pltpu.sync_copy(idx_hbm.at[0, pl.ds(p0 * 2 * CHUNK, CHUNK)], q0.at[0, 0])
    pltpu.sync_copy(idx_hbm.at[0, pl.ds(p0 * 2 * CHUNK + CHUNK, CHUNK)], q0.at[0, 1])
    pltpu.sync_copy(idx_hbm.at[1, pl.ds(p0 * 2 * CHUNK, CHUNK)], q0.at[1, 0])
    pltpu.sync_copy(idx_hbm.at[1, pl.ds(p0 * 2 * CHUNK + CHUNK, CHUNK)], q0.at[1, 1])
    pltpu.async_copy(y_hbm.at[q0.at[0, 0]], rows_a, sem_a)

    def dbody(q, carry):
        j0 = p0 + 2 * q
        halfstep(q0, q1, j0 + 1, sem_i1)
        halfstep(q1, q0, j0 + 2, sem_i0)
        return carry

    lax.fori_loop(0, (PPW - 1) // 2, dbody, 0)

    pltpu.async_copy(y_hbm.at[q0.at[0, 1]], rows_b, sem_b)
    pltpu.make_async_copy(y_hbm.at[q0.at[0, 0]], rows_a, sem_a).wait()
    pltpu.sync_copy(rows_a, acc.at[q0.at[1, 0]], add=True)
    pltpu.make_async_copy(y_hbm.at[q0.at[0, 1]], rows_b, sem_b).wait()
    pltpu.sync_copy(rows_b, acc.at[q0.at[1, 1]], add=True)

    @pl.when(wid < 2)
    def _():
        pe = NW * PPW + wid
        pltpu.sync_copy(idx_hbm.at[0, pl.ds(pe * 2 * CHUNK, CHUNK)], q0.at[0, 0])
        pltpu.sync_copy(idx_hbm.at[0, pl.ds(pe * 2 * CHUNK + CHUNK, CHUNK)], q0.at[0, 1])
        pltpu.sync_copy(idx_hbm.at[1, pl.ds(pe * 2 * CHUNK, CHUNK)], q0.at[1, 0])
        pltpu.sync_copy(idx_hbm.at[1, pl.ds(pe * 2 * CHUNK + CHUNK, CHUNK)], q0.at[1, 1])
        cp_a = pltpu.async_copy(y_hbm.at[q0.at[0, 0]], rows_a, sem_a)
        cp_b = pltpu.async_copy(y_hbm.at[q0.at[0, 1]], rows_b, sem_b)
        cp_a.wait()
        pltpu.sync_copy(rows_a, acc.at[q0.at[1, 0]], add=True)
        cp_b.wait()
        pltpu.sync_copy(rows_b, acc.at[q0.at[1, 1]], add=True)

    plsc.subcore_barrier()
    pltpu.sync_copy(
        acc.at[pl.ds(s * ZSTRIPE, ZSTRIPE)],
        out_hbm.at[c, pl.ds(s * ZSTRIPE, ZSTRIPE)],
    )


def _dinv_mm_body(dg_ref, x_ref, w_ref, y_ref, dinv_ref):
    dg = dg_ref[...]
    d = dg[0, :, 0:1] + dg[1, :, 0:1] + 1.0
    dinvb = jnp.broadcast_to(lax.rsqrt(d), (RBLK, D))
    xw = jnp.dot(x_ref[...], w_ref[...], preferred_element_type=jnp.float32)
    y_ref[...] = xw * dinvb
    dinv_ref[...] = dinvb


_dinv_mm = pl.pallas_call(
    _dinv_mm_body,
    grid=(GRID,),
    in_specs=[
        pl.BlockSpec((NC, RBLK, DEGW), lambda i: (0, i, 0)),
        pl.BlockSpec((RBLK, D), lambda i: (i, 0)),
        pl.BlockSpec((D, D), lambda i: (0, 0)),
    ],
    out_specs=[
        pl.BlockSpec((RBLK, D), lambda i: (i, 0)),
        pl.BlockSpec((RBLK, D), lambda i: (i, 0)),
    ],
    out_shape=[
        jax.ShapeDtypeStruct((N, D), jnp.float32),
        jax.ShapeDtypeStruct((N, D), jnp.float32),
    ],
)


def _layer2_body(z_ref, y_ref, dinv_ref, b_ref, w_ref, o_ref):
    zsum = z_ref[0] + z_ref[1]
    h = jnp.maximum(dinv_ref[...] * (zsum + y_ref[...]) + b_ref[...], 0.0)
    hw = jnp.dot(h, w_ref[...], preferred_element_type=jnp.float32)
    o_ref[...] = hw * dinv_ref[...]


_layer2 = pl.pallas_call(
    _layer2_body,
    grid=(GRID,),
    in_specs=[
        pl.BlockSpec((NC, RBLK, D), lambda i: (0, i, 0)),
        pl.BlockSpec((RBLK, D), lambda i: (i, 0)),
        pl.BlockSpec((RBLK, D), lambda i: (i, 0)),
        pl.BlockSpec((D,), lambda i: (0,)),
        pl.BlockSpec((D, D), lambda i: (0, 0)),
    ],
    out_specs=pl.BlockSpec((RBLK, D), lambda i: (i, 0)),
    out_shape=jax.ShapeDtypeStruct((N, D), jnp.float32),
)


def _final_body(z_ref, y_ref, dinv_ref, b_ref, o_ref):
    o = dinv_ref[...] * (z_ref[0] + z_ref[1] + y_ref[...]) + b_ref[...]
    m = jnp.max(o, axis=1, keepdims=True)
    t = o - m
    o_ref[...] = t - jnp.log(jnp.sum(jnp.exp(t), axis=1, keepdims=True))


_final = pl.pallas_call(
    _final_body,
    grid=(GRID,),
    in_specs=[
        pl.BlockSpec((NC, RBLK, D), lambda i: (0, i, 0)),
        pl.BlockSpec((RBLK, D), lambda i: (i, 0)),
        pl.BlockSpec((RBLK, D), lambda i: (i, 0)),
        pl.BlockSpec((D,), lambda i: (0,)),
    ],
    out_specs=pl.BlockSpec((RBLK, D), lambda i: (i, 0)),
    out_shape=jax.ShapeDtypeStruct((N, D), jnp.float32),
)


def kernel(x, edge_index, W1, b1, W2, b2):
    idxp = edge_index.astype(jnp.int32)
    ones_rows = jnp.ones((CHUNK, DEGW), jnp.float32)
    zer_d = jnp.zeros((CHUNK, D), jnp.float32)

    degp = _deg_kernel(idxp, ones_rows, zer_d)
    y1, dinvb = _dinv_mm(degp, x, W1)
    z1 = _edge_kernel(y1, idxp, zer_d)
    y2 = _layer2(z1, y1, dinvb, b1, W2)
    z2 = _edge_kernel(y2, idxp, zer_d)
    return _final(z2, y2, dinvb, b2)


# speedup vs baseline: 1.0119x; 1.0119x over previous
import functools

import jax
import jax.numpy as jnp
from jax import lax
from jax.experimental import pallas as pl
from jax.experimental.pallas import tpu as pltpu
from jax.experimental.pallas import tpu_sc as plsc

N = 10000
E = 320000
D = 128
NC = 2
NS = 16
NW = NC * NS
CHUNK = 128
ROWS = E // CHUNK
PAIRS = ROWS // 2
PPW = PAIRS // NW
ZROWS = 10112
ZSTRIPE = ZROWS // NS
DEGW = 128
RBLK = 5000
GRID = N // RBLK

_sc_mesh = plsc.VectorSubcoreMesh(
    core_axis_name="c", subcore_axis_name="s", num_cores=NC, num_subcores=NS
)


def _init_stripe(zer_hbm, zbuf, acc, s):
    pltpu.sync_copy(zer_hbm, zbuf)
    for i in range(ZSTRIPE // CHUNK):
        pltpu.sync_copy(zbuf, acc.at[pl.ds(s * ZSTRIPE + i * CHUNK, CHUNK)])
    rem = ZSTRIPE % CHUNK
    if rem:
        pltpu.sync_copy(
            zbuf.at[pl.ds(0, rem)],
            acc.at[pl.ds(s * ZSTRIPE + ZSTRIPE - rem, rem)],
        )


@functools.partial(
    pl.kernel,
    out_type=jax.ShapeDtypeStruct((NC, ZROWS, DEGW), jnp.float32),
    mesh=_sc_mesh,
    scratch_types=[
        pltpu.VMEM((2, CHUNK), jnp.int32),
        pltpu.VMEM((2, CHUNK), jnp.int32),
        pltpu.VMEM((CHUNK, DEGW), jnp.float32),
        pltpu.VMEM((CHUNK, DEGW), jnp.float32),
        pltpu.VMEM_SHARED((ZROWS, DEGW), jnp.float32),
        pltpu.SemaphoreType.DMA,
    ],
)
def _deg_kernel(idx_hbm, ones_hbm, zer_hbm, out_hbm,
                q0, q1, onesv, zbuf, acc, sem_i):
    c = lax.axis_index("c")
    s = lax.axis_index("s")
    wid = c * NS + s
    _init_stripe(zer_hbm, zbuf, acc, s)
    pltpu.sync_copy(ones_hbm, onesv)
    plsc.subcore_barrier()

    p0 = wid * PPW

    def load(p, q):
        pltpu.async_copy(idx_hbm.at[1, pl.ds(p * 2 * CHUNK, CHUNK)],
                         q.at[0], sem_i)
        pltpu.async_copy(idx_hbm.at[1, pl.ds(p * 2 * CHUNK + CHUNK, CHUNK)],
                         q.at[1], sem_i)

    def drain(p, q):
        pltpu.make_async_copy(idx_hbm.at[1, pl.ds(p * 2 * CHUNK, CHUNK)],
                              q.at[0], sem_i).wait()
        pltpu.make_async_copy(idx_hbm.at[1, pl.ds(p * 2 * CHUNK, CHUNK)],
                              q.at[1], sem_i).wait()

    def scat(q):
        pltpu.sync_copy(onesv, acc.at[q.at[0]], add=True)
        pltpu.sync_copy(onesv, acc.at[q.at[1]], add=True)

    load(p0, q0)
    drain(p0, q0)

    def dbody(k, carry):
        j0 = p0 + 2 * k
        load(j0 + 1, q1)
        scat(q0)
        drain(j0 + 1, q1)
        load(j0 + 2, q0)
        scat(q1)
        drain(j0 + 2, q0)
        return carry

    lax.fori_loop(0, (PPW - 1) // 2, dbody, 0)
    scat(q0)

    @pl.when(wid < 2)
    def _():
        load(NW * PPW + wid, q1)
        drain(NW * PPW + wid, q1)
        scat(q1)

    plsc.subcore_barrier()
    pltpu.sync_copy(
        acc.at[pl.ds(s * ZSTRIPE, ZSTRIPE)],
        out_hbm.at[c, pl.ds(s * ZSTRIPE, ZSTRIPE)],
    )


@functools.partial(
    pl.kernel,
    out_type=jax.ShapeDtypeStruct((NC, ZROWS, D), jnp.float32),
    mesh=_sc_mesh,
    scratch_types=[
        pltpu.VMEM((2, 2, CHUNK), jnp.int32),
        pltpu.VMEM((2, 2, CHUNK), jnp.int32),
        pltpu.VMEM((CHUNK, D), jnp.float32),
        pltpu.VMEM((CHUNK, D), jnp.float32),
        pltpu.VMEM_SHARED((ZROWS, D), jnp.float32),
        pltpu.SemaphoreType.DMA,
        pltpu.SemaphoreType.DMA,
        pltpu.SemaphoreType.DMA,
        pltpu.SemaphoreType.DMA,
    ],
)
def _edge_kernel(y_hbm, idx_hbm, zer_hbm, out_hbm,
                 q0, q1, rows_a, rows_b, acc, sem_a, sem_b, sem_i1, sem_i0):
    c = lax.axis_index("c")
    s = lax.axis_index("s")
    wid = c * NS + s
    _init_stripe(zer_hbm, rows_a, acc, s)
    plsc.subcore_barrier()

    p0 = wid * PPW

    def halfstep(qc, qn, pn, sem_in):
        pltpu.async_copy(y_hbm.at[qc.at[0, 1]], rows_b, sem_b)
        pltpu.async_copy(idx_hbm.at[0, pl.ds(pn * 2 * CHUNK, CHUNK)],
                         qn.at[0, 0], sem_in)
        pltpu.async_copy(idx_hbm.at[0, pl.ds(pn * 2 * CHUNK + CHUNK, CHUNK)],
                         qn.at[0, 1], sem_in)
        pltpu.async_copy(idx_hbm.at[1, pl.ds(pn * 2 * CHUNK, CHUNK)],
                         qn.at[1, 0], sem_in)
        pltpu.async_copy(idx_hbm.at[1, pl.ds(pn * 2 * CHUNK + CHUNK, CHUNK)],
                         qn.at[1, 1], sem_in)
        pltpu.make_async_copy(y_hbm.at[qc.at[0, 0]], rows_a, sem_a).wait()
        pltpu.sync_copy(rows_a, acc.at[qc.at[1, 0]], add=True)
        for _k in range(4):
            pltpu.make_async_copy(idx_hbm.at[0, pl.ds(pn * 2 * CHUNK, CHUNK)],
                                  qn.at[0, 0], sem_in).wait()
        pltpu.async_copy(y_hbm.at[qn.at[0, 0]], rows_a, sem_a)
        pltpu.make_async_copy(y_hbm.at[qc.at[0, 1]], rows_b, sem_b).wait()
        pltpu.sync_copy(rows_b, acc.at[qc.at[1, 1]], add=True)

    pltpu.sync_copy(idx_hbm.at[0, pl.ds(p0 * 2 * CHUNK, CHUNK)], q0.at[0, 0])
    pltpu.sync_copy(idx_hbm.at[0, pl.ds(p0 * 2 * CHUNK + CHUNK, CHUNK)], q0.at[0, 1])
    pltpu.sync_copy(idx_hbm.at[1, pl.ds(p0 * 2 * CHUNK, CHUNK)], q0.at[1, 0])
    pltpu.sync_copy(idx_hbm.at[1, pl.ds(p0 * 2 * CHUNK + CHUNK, CHUNK)], q0.at[1, 1])
    pltpu.async_copy(y_hbm.at[q0.at[0, 0]], rows_a, sem_a)

    def dbody(q, carry):
        j0 = p0 + 2 * q
        halfstep(q0, q1, j0 + 1, sem_i1)
        halfstep(q1, q0, j0 + 2, sem_i0)
        return carry

    lax.fori_loop(0, (PPW - 1) // 2, dbody, 0)

    pltpu.async_copy(y_hbm.at[q0.at[0, 1]], rows_b, sem_b)
    pltpu.make_async_copy(y_hbm.at[q0.at[0, 0]], rows_a, sem_a).wait()
    pltpu.sync_copy(rows_a, acc.at[q0.at[1, 0]], add=True)
    pltpu.make_async_copy(y_hbm.at[q0.at[0, 1]], rows_b, sem_b).wait()
    pltpu.sync_copy(rows_b, acc.at[q0.at[1, 1]], add=True)

    @pl.when(wid < 2)
    def _():
        pe = NW * PPW + wid
        pltpu.sync_copy(idx_hbm.at[0, pl.ds(pe * 2 * CHUNK, CHUNK)], q0.at[0, 0])
        pltpu.sync_copy(idx_hbm.at[0, pl.ds(pe * 2 * CHUNK + CHUNK, CHUNK)], q0.at[0, 1])
        pltpu.sync_copy(idx_hbm.at[1, pl.ds(pe * 2 * CHUNK, CHUNK)], q0.at[1, 0])
        pltpu.sync_copy(idx_hbm.at[1, pl.ds(pe * 2 * CHUNK + CHUNK, CHUNK)], q0.at[1, 1])
        cp_a = pltpu.async_copy(y_hbm.at[q0.at[0, 0]], rows_a, sem_a)
        cp_b = pltpu.async_copy(y_hbm.at[q0.at[0, 1]], rows_b, sem_b)
        cp_a.wait()
        pltpu.sync_copy(rows_a, acc.at[q0.at[1, 0]], add=True)
        cp_b.wait()
        pltpu.sync_copy(rows_b, acc.at[q0.at[1, 1]], add=True)

    plsc.subcore_barrier()
    pltpu.sync_copy(
        acc.at[pl.ds(s * ZSTRIPE, ZSTRIPE)],
        out_hbm.at[c, pl.ds(s * ZSTRIPE, ZSTRIPE)],
    )


def _dinv_mm_body(dg_ref, x_ref, w_ref, y_ref, dinv_ref):
    dg = dg_ref[...]
    d = dg[0, :, 0:1] + dg[1, :, 0:1] + 1.0
    dinvb = jnp.broadcast_to(lax.rsqrt(d), (RBLK, D))
    xw = jnp.dot(x_ref[...], w_ref[...], preferred_element_type=jnp.float32)
    y_ref[...] = xw * dinvb
    dinv_ref[...] = dinvb


_dinv_mm = pl.pallas_call(
    _dinv_mm_body,
    grid=(GRID,),
    in_specs=[
        pl.BlockSpec((NC, RBLK, DEGW), lambda i: (0, i, 0)),
        pl.BlockSpec((RBLK, D), lambda i: (i, 0)),
        pl.BlockSpec((D, D), lambda i: (0, 0)),
    ],
    out_specs=[
        pl.BlockSpec((RBLK, D), lambda i: (i, 0)),
        pl.BlockSpec((RBLK, D), lambda i: (i, 0)),
    ],
    out_shape=[
        jax.ShapeDtypeStruct((N, D), jnp.float32),
        jax.ShapeDtypeStruct((N, D), jnp.float32),
    ],
)


def _layer2_body(z_ref, y_ref, dinv_ref, b_ref, w_ref, o_ref):
    zsum = z_ref[0] + z_ref[1]
    h = jnp.maximum(dinv_ref[...] * (zsum + y_ref[...]) + b_ref[...], 0.0)
    hw = jnp.dot(h, w_ref[...], preferred_element_type=jnp.float32)
    o_ref[...] = hw * dinv_ref[...]


_layer2 = pl.pallas_call(
    _layer2_body,
    grid=(GRID,),
    in_specs=[
        pl.BlockSpec((NC, RBLK, D), lambda i: (0, i, 0)),
        pl.BlockSpec((RBLK, D), lambda i: (i, 0)),
        pl.BlockSpec((RBLK, D), lambda i: (i, 0)),
        pl.BlockSpec((D,), lambda i: (0,)),
        pl.BlockSpec((D, D), lambda i: (0, 0)),
    ],
    out_specs=pl.BlockSpec((RBLK, D), lambda i: (i, 0)),
    out_shape=jax.ShapeDtypeStruct((N, D), jnp.float32),
)


def _final_body(z_ref, y_ref, dinv_ref, b_ref, o_ref):
    o = dinv_ref[...] * (z_ref[0] + z_ref[1] + y_ref[...]) + b_ref[...]
    m = jnp.max(o, axis=1, keepdims=True)
    t = o - m
    o_ref[...] = t - jnp.log(jnp.sum(jnp.exp(t), axis=1, keepdims=True))


_final = pl.pallas_call(
    _final_body,
    grid=(GRID,),
    in_specs=[
        pl.BlockSpec((NC, RBLK, D), lambda i: (0, i, 0)),
        pl.BlockSpec((RBLK, D), lambda i: (i, 0)),
        pl.BlockSpec((RBLK, D), lambda i: (i, 0)),
        pl.BlockSpec((D,), lambda i: (0,)),
    ],
    out_specs=pl.BlockSpec((RBLK, D), lambda i: (i, 0)),
    out_shape=jax.ShapeDtypeStruct((N, D), jnp.float32),
)


def kernel(x, edge_index, W1, b1, W2, b2):
    idxp = edge_index.astype(jnp.int32)
    ones_rows = jnp.ones((CHUNK, DEGW), jnp.float32)
    zer_d = jnp.zeros((CHUNK, D), jnp.float32)

    degp = _deg_kernel(idxp, ones_rows, zer_d)
    y1, dinvb = _dinv_mm(degp, x, W1)
    z1 = _edge_kernel(y1, idxp, zer_d)
    y2 = _layer2(z1, y1, dinvb, b1, W2)
    z2 = _edge_kernel(y2, idxp, zer_d)
    return _final(z2, y2, dinvb, b2)


# speedup vs baseline: 1.0246x; 1.0125x over previous
import functools

import jax
import jax.numpy as jnp
from jax import lax
from jax.experimental import pallas as pl
from jax.experimental.pallas import tpu as pltpu
from jax.experimental.pallas import tpu_sc as plsc

N = 10000
E = 320000
D = 128
NC = 2
NS = 16
NW = NC * NS
CHUNK = 128
ROWS = E // CHUNK
PAIRS = ROWS // 2
PPW = PAIRS // NW
ZROWS = 10112
ZSTRIPE = ZROWS // NS
DEGW = 128
RBLK = 5000
GRID = N // RBLK

_sc_mesh = plsc.VectorSubcoreMesh(
    core_axis_name="c", subcore_axis_name="s", num_cores=NC, num_subcores=NS
)


def _init_stripe(zer_hbm, zbuf, acc, s):
    pltpu.sync_copy(zer_hbm, zbuf)
    for i in range(ZSTRIPE // CHUNK):
        pltpu.sync_copy(zbuf, acc.at[pl.ds(s * ZSTRIPE + i * CHUNK, CHUNK)])
    rem = ZSTRIPE % CHUNK
    if rem:
        pltpu.sync_copy(
            zbuf.at[pl.ds(0, rem)],
            acc.at[pl.ds(s * ZSTRIPE + ZSTRIPE - rem, rem)],
        )


@functools.partial(
    pl.kernel,
    out_type=jax.ShapeDtypeStruct((NC, ZROWS, DEGW), jnp.float32),
    mesh=_sc_mesh,
    scratch_types=[
        pltpu.VMEM((2, CHUNK), jnp.int32),
        pltpu.VMEM((2, CHUNK), jnp.int32),
        pltpu.VMEM((CHUNK, DEGW), jnp.float32),
        pltpu.VMEM((CHUNK, DEGW), jnp.float32),
        pltpu.VMEM_SHARED((ZROWS, DEGW), jnp.float32),
        pltpu.SemaphoreType.DMA,
    ],
)
def _deg_kernel(idx_hbm, ones_hbm, zer_hbm, out_hbm,
                q0, q1, onesv, zbuf, acc, sem_i):
    c = lax.axis_index("c")
    s = lax.axis_index("s")
    wid = c * NS + s
    p0 = wid * PPW

    def load0(p, q):
        pltpu.async_copy(idx_hbm.at[1, pl.ds(p * 2 * CHUNK, CHUNK)],
                         q.at[0], sem_i)
        pltpu.async_copy(idx_hbm.at[1, pl.ds(p * 2 * CHUNK + CHUNK, CHUNK)],
                         q.at[1], sem_i)

    load0(p0, q0)
    _init_stripe(zer_hbm, zbuf, acc, s)
    pltpu.sync_copy(ones_hbm, onesv)
    plsc.subcore_barrier()

    def load(p, q):
        pltpu.async_copy(idx_hbm.at[1, pl.ds(p * 2 * CHUNK, CHUNK)],
                         q.at[0], sem_i)
        pltpu.async_copy(idx_hbm.at[1, pl.ds(p * 2 * CHUNK + CHUNK, CHUNK)],
                         q.at[1], sem_i)

    def drain(p, q):
        pltpu.make_async_copy(idx_hbm.at[1, pl.ds(p * 2 * CHUNK, CHUNK)],
                              q.at[0], sem_i).wait()
        pltpu.make_async_copy(idx_hbm.at[1, pl.ds(p * 2 * CHUNK, CHUNK)],
                              q.at[1], sem_i).wait()

    def scat(q):
        pltpu.sync_copy(onesv, acc.at[q.at[0]], add=True)
        pltpu.sync_copy(onesv, acc.at[q.at[1]], add=True)

    drain(p0, q0)

    def dbody(k, carry):
        j0 = p0 + 2 * k
        load(j0 + 1, q1)
        scat(q0)
        drain(j0 + 1, q1)
        load(j0 + 2, q0)
        scat(q1)
        drain(j0 + 2, q0)
        return carry

    lax.fori_loop(0, (PPW - 1) // 2, dbody, 0)
    scat(q0)

    @pl.when(wid < 2)
    def _():
        load(NW * PPW + wid, q1)
        drain(NW * PPW + wid, q1)
        scat(q1)

    plsc.subcore_barrier()
    pltpu.sync_copy(
        acc.at[pl.ds(s * ZSTRIPE, ZSTRIPE)],
        out_hbm.at[c, pl.ds(s * ZSTRIPE, ZSTRIPE)],
    )


@functools.partial(
    pl.kernel,
    out_type=jax.ShapeDtypeStruct((NC, ZROWS, D), jnp.float32),
    mesh=_sc_mesh,
    scratch_types=[
        pltpu.VMEM((2, 2, CHUNK), jnp.int32),
        pltpu.VMEM((2, 2, CHUNK), jnp.int32),
        pltpu.VMEM((CHUNK, D), jnp.float32),
        pltpu.VMEM((CHUNK, D), jnp.float32),
        pltpu.VMEM((80, D), jnp.float32),
        pltpu.VMEM_SHARED((ZROWS, D), jnp.float32),
        pltpu.SemaphoreType.DMA,
        pltpu.SemaphoreType.DMA,
        pltpu.SemaphoreType.DMA,
        pltpu.SemaphoreType.DMA,
    ],
)
def _edge_kernel(y_hbm, idx_hbm, zer_hbm, out_hbm,
                 q0, q1, rows_a, rows_b, zbuf, acc, sem_a, sem_b, sem_i1, sem_i0):
    c = lax.axis_index("c")
    s = lax.axis_index("s")
    wid = c * NS + s
    p0 = wid * PPW
    pltpu.sync_copy(idx_hbm.at[0, pl.ds(p0 * 2 * CHUNK, CHUNK)], q0.at[0, 0])
    pltpu.sync_copy(idx_hbm.at[0, pl.ds(p0 * 2 * CHUNK + CHUNK, CHUNK)], q0.at[0, 1])
    pltpu.sync_copy(idx_hbm.at[1, pl.ds(p0 * 2 * CHUNK, CHUNK)], q0.at[1, 0])
    pltpu.sync_copy(idx_hbm.at[1, pl.ds(p0 * 2 * CHUNK + CHUNK, CHUNK)], q0.at[1, 1])
    pltpu.async_copy(y_hbm.at[q0.at[0, 0]], rows_a, sem_a)
    pltpu.sync_copy(zer_hbm.at[pl.ds(0, 80)], zbuf)
    for i in range(8):
        pltpu.sync_copy(zbuf.at[pl.ds(0, 79)],
                        acc.at[pl.ds(s * ZSTRIPE + i * 79, 79)])
    plsc.subcore_barrier()

    def halfstep(qc, qn, pn, sem_in):
        pltpu.async_copy(y_hbm.at[qc.at[0, 1]], rows_b, sem_b)
        pltpu.async_copy(idx_hbm.at[0, pl.ds(pn * 2 * CHUNK, CHUNK)],
                         qn.at[0, 0], sem_in)
        pltpu.async_copy(idx_hbm.at[0, pl.ds(pn * 2 * CHUNK + CHUNK, CHUNK)],
                         qn.at[0, 1], sem_in)
        pltpu.async_copy(idx_hbm.at[1, pl.ds(pn * 2 * CHUNK, CHUNK)],
                         qn.at[1, 0], sem_in)
        pltpu.async_copy(idx_hbm.at[1, pl.ds(pn * 2 * CHUNK + CHUNK, CHUNK)],
                         qn.at[1, 1], sem_in)
        pltpu.make_async_copy(y_hbm.at[qc.at[0, 0]], rows_a, sem_a).wait()
        pltpu.sync_copy(rows_a, acc.at[qc.at[1, 0]], add=True)
        for _k in range(4):
            pltpu.make_async_copy(idx_hbm.at[0, pl.ds(pn * 2 * CHUNK, CHUNK)],
                                  qn.at[0, 0], sem_in).wait()
        pltpu.async_copy(y_hbm.at[qn.at[0, 0]], rows_a, sem_a)
        pltpu.make_async_copy(y_hbm.at[qc.at[0, 1]], rows_b, sem_b).wait()
        pltpu.sync_copy(rows_b, acc.at[qc.at[1, 1]], add=True)

    def dbody(q, carry):
        j0 = p0 + 2 * q
        halfstep(q0, q1, j0 + 1, sem_i1)
        halfstep(q1, q0, j0 + 2, sem_i0)
        return carry

    lax.fori_loop(0, (PPW - 1) // 2, dbody, 0)

    pltpu.async_copy(y_hbm.at[q0.at[0, 1]], rows_b, sem_b)
    pltpu.make_async_copy(y_hbm.at[q0.at[0, 0]], rows_a, sem_a).wait()
    pltpu.sync_copy(rows_a, acc.at[q0.at[1, 0]], add=True)
    pltpu.make_async_copy(y_hbm.at[q0.at[0, 1]], rows_b, sem_b).wait()
    pltpu.sync_copy(rows_b, acc.at[q0.at[1, 1]], add=True)

    @pl.when(wid < 2)
    def _():
        pe = NW * PPW + wid
        pltpu.sync_copy(idx_hbm.at[0, pl.ds(pe * 2 * CHUNK, CHUNK)], q0.at[0, 0])
        pltpu.sync_copy(idx_hbm.at[0, pl.ds(pe * 2 * CHUNK + CHUNK, CHUNK)], q0.at[0, 1])
        pltpu.sync_copy(idx_hbm.at[1, pl.ds(pe * 2 * CHUNK, CHUNK)], q0.at[1, 0])
        pltpu.sync_copy(idx_hbm.at[1, pl.ds(pe * 2 * CHUNK + CHUNK, CHUNK)], q0.at[1, 1])
        cp_a = pltpu.async_copy(y_hbm.at[q0.at[0, 0]], rows_a, sem_a)
        cp_b = pltpu.async_copy(y_hbm.at[q0.at[0, 1]], rows_b, sem_b)
        cp_a.wait()
        pltpu.sync_copy(rows_a, acc.at[q0.at[1, 0]], add=True)
        cp_b.wait()
        pltpu.sync_copy(rows_b, acc.at[q0.at[1, 1]], add=True)

    plsc.subcore_barrier()
    pltpu.sync_copy(
        acc.at[pl.ds(s * ZSTRIPE, ZSTRIPE)],
        out_hbm.at[c, pl.ds(s * ZSTRIPE, ZSTRIPE)],
    )


def _dinv_mm_body(dg_ref, x_ref, w_ref, y_ref, dinv_ref):
    dg = dg_ref[...]
    d = dg[0, :, 0:1] + dg[1, :, 0:1] + 1.0
    dinvb = jnp.broadcast_to(lax.rsqrt(d), (RBLK, D))
    xw = jnp.dot(x_ref[...], w_ref[...], preferred_element_type=jnp.float32)
    y_ref[...] = xw * dinvb
    dinv_ref[...] = dinvb


_dinv_mm = pl.pallas_call(
    _dinv_mm_body,
    grid=(GRID,),
    in_specs=[
        pl.BlockSpec((NC, RBLK, DEGW), lambda i: (0, i, 0)),
        pl.BlockSpec((RBLK, D), lambda i: (i, 0)),
        pl.BlockSpec((D, D), lambda i: (0, 0)),
    ],
    out_specs=[
        pl.BlockSpec((RBLK, D), lambda i: (i, 0)),
        pl.BlockSpec((RBLK, D), lambda i: (i, 0)),
    ],
    out_shape=[
        jax.ShapeDtypeStruct((N, D), jnp.float32),
        jax.ShapeDtypeStruct((N, D), jnp.float32),
    ],
)


def _layer2_body(z_ref, y_ref, dinv_ref, b_ref, w_ref, o_ref):
    zsum = z_ref[0] + z_ref[1]
    h = jnp.maximum(dinv_ref[...] * (zsum + y_ref[...]) + b_ref[...], 0.0)
    hw = jnp.dot(h, w_ref[...], preferred_element_type=jnp.float32)
    o_ref[...] = hw * dinv_ref[...]


_layer2 = pl.pallas_call(
    _layer2_body,
    grid=(GRID,),
    in_specs=[
        pl.BlockSpec((NC, RBLK, D), lambda i: (0, i, 0)),
        pl.BlockSpec((RBLK, D), lambda i: (i, 0)),
        pl.BlockSpec((RBLK, D), lambda i: (i, 0)),
        pl.BlockSpec((D,), lambda i: (0,)),
        pl.BlockSpec((D, D), lambda i: (0, 0)),
    ],
    out_specs=pl.BlockSpec((RBLK, D), lambda i: (i, 0)),
    out_shape=jax.ShapeDtypeStruct((N, D), jnp.float32),
)


def _final_body(z_ref, y_ref, dinv_ref, b_ref, o_ref):
    o = dinv_ref[...] * (z_ref[0] + z_ref[1] + y_ref[...]) + b_ref[...]
    m = jnp.max(o, axis=1, keepdims=True)
    t = o - m
    o_ref[...] = t - jnp.log(jnp.sum(jnp.exp(t), axis=1, keepdims=True))


_final = pl.pallas_call(
    _final_body,
    grid=(GRID,),
    in_specs=[
        pl.BlockSpec((NC, RBLK, D), lambda i: (0, i, 0)),
        pl.BlockSpec((RBLK, D), lambda i: (i, 0)),
        pl.BlockSpec((RBLK, D), lambda i: (i, 0)),
        pl.BlockSpec((D,), lambda i: (0,)),
    ],
    out_specs=pl.BlockSpec((RBLK, D), lambda i: (i, 0)),
    out_shape=jax.ShapeDtypeStruct((N, D), jnp.float32),
)


def kernel(x, edge_index, W1, b1, W2, b2):
    idxp = edge_index.astype(jnp.int32)
    ones_rows = jnp.ones((CHUNK, DEGW), jnp.float32)
    zer_d = jnp.zeros((CHUNK, D), jnp.float32)

    degp = _deg_kernel(idxp, ones_rows, zer_d)
    y1, dinvb = _dinv_mm(degp, x, W1)
    z1 = _edge_kernel(y1, idxp, zer_d)
    y2 = _layer2(z1, y1, dinvb, b1, W2)
    z2 = _edge_kernel(y2, idxp, zer_d)
    return _final(z2, y2, dinvb, b2)
